# Initial kernel scaffold; baseline (speedup 1.0000x reference)
import jax
import jax.numpy as jnp
from jax.experimental import pallas as pl


def kernel(x, edge_index, edge_values, W0, b):
    raise NotImplementedError("write your pallas kernel here")


# speedup vs baseline: 4.0468x; 4.0468x over previous
import functools

import jax
import jax.numpy as jnp
from jax import lax
from jax.experimental import pallas as pl
from jax.experimental.pallas import tpu as pltpu
from jax.experimental.pallas import tpu_sc as plsc

N = 10000
E = 320000
D = 128

NC = 2
NS = 16
NW = NC * NS
EPW = E // NW
CHUNK = 80
NCHUNK = EPW // CHUNK
ROWS_PT = N // NS


def _sc_body(pre_hbm, rows_hbm, cols_hbm, vals_hbm, zeros_hbm, out_hbm,
             accum, rows_idx, cols_idx, vals_v, rows_buf, sem):
    c = lax.axis_index("c")
    s = lax.axis_index("s")
    w = c * NS + s

    r0 = pl.multiple_of(s * ROWS_PT, 8)
    pltpu.sync_copy(zeros_hbm.at[pl.ds(r0, ROWS_PT), :],
                    accum.at[pl.ds(r0, ROWS_PT), :])
    plsc.subcore_barrier()

    base0 = w * EPW

    def chunk_body(j, carry):
        base = pl.multiple_of(base0 + j * CHUNK, 8)
        pltpu.sync_copy(rows_hbm.at[pl.ds(base, CHUNK)], rows_idx)
        pltpu.sync_copy(cols_hbm.at[pl.ds(base, CHUNK)], cols_idx)
        pltpu.sync_copy(vals_hbm.at[pl.ds(base, CHUNK)], vals_v)
        pltpu.async_copy(pre_hbm.at[cols_idx], rows_buf, sem).wait()

        def edge_body(e, carry2):
            ev = plsc.load_gather(vals_v, [jnp.full((16,), e, jnp.int32)])
            for q in range(D // 16):
                sl = pl.ds(q * 16, 16)
                rows_buf[e, sl] = rows_buf[e, sl] * ev
            return carry2

        lax.fori_loop(0, CHUNK, edge_body, 0, unroll=2)

        pltpu.sync_copy(rows_buf, accum.at[rows_idx], add=True)
        return carry

    lax.fori_loop(0, NCHUNK, chunk_body, 0)

    plsc.subcore_barrier()
    pltpu.sync_copy(accum.at[pl.ds(r0, ROWS_PT), :],
                    out_hbm.at[c, pl.ds(r0, ROWS_PT), :])


_sc_scatter = functools.partial(
    pl.kernel,
    out_type=jax.ShapeDtypeStruct((NC, N, D), jnp.float32),
    mesh=plsc.VectorSubcoreMesh(core_axis_name="c", subcore_axis_name="s"),
    scratch_types=[
        pltpu.VMEM_SHARED((N, D), jnp.float32),
        pltpu.VMEM((CHUNK,), jnp.int32),
        pltpu.VMEM((CHUNK,), jnp.int32),
        pltpu.VMEM((CHUNK,), jnp.float32),
        pltpu.VMEM((CHUNK, D), jnp.float32),
        pltpu.SemaphoreType.DMA,
    ],
    compiler_params=pltpu.CompilerParams(use_tc_tiling_on_sc=False,
                                         needs_layout_passes=False),
)(_sc_body)


def _mm_body(x_ref, w_ref, o_ref):
    o_ref[...] = jnp.dot(x_ref[...], w_ref[...],
                         preferred_element_type=jnp.float32)


def _fin_body(p_ref, b_ref, o_ref):
    o_ref[...] = jnp.maximum(p_ref[0] + p_ref[1] + b_ref[...], 0.0)


_MM_ROWS = 1000


def kernel(x, edge_index, edge_values, W0, b):
    pre = pl.pallas_call(
        _mm_body,
        grid=(N // _MM_ROWS,),
        in_specs=[
            pl.BlockSpec((_MM_ROWS, D), lambda i: (i, 0)),
            pl.BlockSpec((D, D), lambda i: (0, 0)),
        ],
        out_specs=pl.BlockSpec((_MM_ROWS, D), lambda i: (i, 0)),
        out_shape=jax.ShapeDtypeStruct((N, D), jnp.float32),
    )(x, W0)

    zeros = jnp.zeros((N, D), jnp.float32)
    parts = _sc_scatter(pre, edge_index[0], edge_index[1], edge_values, zeros)

    return pl.pallas_call(
        _fin_body,
        grid=(N // _MM_ROWS,),
        in_specs=[
            pl.BlockSpec((NC, _MM_ROWS, D), lambda i: (0, i, 0)),
            pl.BlockSpec((1, D), lambda i: (0, 0)),
        ],
        out_specs=pl.BlockSpec((_MM_ROWS, D), lambda i: (i, 0)),
        out_shape=jax.ShapeDtypeStruct((N, D), jnp.float32),
    )(parts, b.reshape(1, D))


# speedup vs baseline: 4.7575x; 1.1756x over previous
import functools

import jax
import jax.numpy as jnp
from jax import lax
from jax.experimental import pallas as pl
from jax.experimental.pallas import tpu as pltpu
from jax.experimental.pallas import tpu_sc as plsc

N = 10000
E = 320000
D = 128

NC = 2
NS = 16
NW = NC * NS
EPW = E // NW
CHUNK = 64
NCHUNK = -(-EPW // CHUNK)
EPW_PAD = NCHUNK * CHUNK
ROWS_PT = N // NS


def _sc_body(pre_hbm, rows_hbm, cols_hbm, vals_hbm, zeros_hbm, out_hbm,
             accum, rows_e, cols_e, vals_e, gb0,
             cidx0, ridx0, vbuf0, gsem0):
    c = lax.axis_index("c")
    s = lax.axis_index("s")
    w = c * NS + s

    r0 = pl.multiple_of(s * ROWS_PT, 8)
    pltpu.sync_copy(zeros_hbm.at[pl.ds(r0, ROWS_PT), :],
                    accum.at[pl.ds(r0, ROWS_PT), :])

    pltpu.sync_copy(rows_hbm.at[w], rows_e)
    pltpu.sync_copy(cols_hbm.at[w], cols_e)
    pltpu.sync_copy(vals_hbm.at[w], vals_e)
    plsc.subcore_barrier()

    def copy_chunk(src2d, j, dst):
        for q in range(CHUNK // 16):
            sl = pl.ds(q * 16, 16)
            dst[sl] = src2d[j, sl]

    def scale(buf, vbuf):
        def edge_body(e, carry):
            ev = plsc.load_gather(vbuf, [jnp.full((16,), e, jnp.int32)])
            for q in range(D // 16):
                sl = pl.ds(q * 16, 16)
                buf[e, sl] = buf[e, sl] * ev
            return carry
        lax.fori_loop(0, CHUNK, edge_body, 0, unroll=4)

    @pl.loop(0, NCHUNK)
    def pipeline(j):
        copy_chunk(cols_e, j, cidx0)
        copy_chunk(rows_e, j, ridx0)
        copy_chunk(vals_e, j, vbuf0)
        pltpu.async_copy(pre_hbm.at[cidx0], gb0, gsem0).wait()
        scale(gb0, vbuf0)
        pltpu.sync_copy(gb0, accum.at[ridx0], add=True)

    plsc.subcore_barrier()
    pltpu.sync_copy(accum.at[pl.ds(r0, ROWS_PT), :],
                    out_hbm.at[c, pl.ds(r0, ROWS_PT), :])


_sc_scatter = functools.partial(
    pl.kernel,
    out_type=jax.ShapeDtypeStruct((NC, N, D), jnp.float32),
    mesh=plsc.VectorSubcoreMesh(core_axis_name="c", subcore_axis_name="s"),
    scratch_types=[
        pltpu.VMEM_SHARED((N, D), jnp.float32),
        pltpu.VMEM((NCHUNK, CHUNK), jnp.int32),
        pltpu.VMEM((NCHUNK, CHUNK), jnp.int32),
        pltpu.VMEM((NCHUNK, CHUNK), jnp.float32),
        pltpu.VMEM((CHUNK, D), jnp.float32),
        pltpu.VMEM((CHUNK,), jnp.int32),
        pltpu.VMEM((CHUNK,), jnp.int32),
        pltpu.VMEM((CHUNK,), jnp.float32),
        pltpu.SemaphoreType.DMA,
    ],
    compiler_params=pltpu.CompilerParams(use_tc_tiling_on_sc=False,
                                         needs_layout_passes=False),
)(_sc_body)


def _mm_body(x_ref, w_ref, o_ref):
    o_ref[...] = jnp.dot(x_ref[...], w_ref[...],
                         preferred_element_type=jnp.float32)


def _fin_body(p_ref, b_ref, o_ref):
    o_ref[...] = jnp.maximum(p_ref[0] + p_ref[1] + b_ref[...], 0.0)


_MM_ROWS = 1000


def _pad_edges(a, dtype):
    a = a.reshape(NW, EPW)
    pad = jnp.zeros((NW, EPW_PAD - EPW), dtype)
    return jnp.concatenate([a, pad], axis=1).reshape(NW, NCHUNK, CHUNK)


def kernel(x, edge_index, edge_values, W0, b):
    pre = pl.pallas_call(
        _mm_body,
        grid=(N // _MM_ROWS,),
        in_specs=[
            pl.BlockSpec((_MM_ROWS, D), lambda i: (i, 0)),
            pl.BlockSpec((D, D), lambda i: (0, 0)),
        ],
        out_specs=pl.BlockSpec((_MM_ROWS, D), lambda i: (i, 0)),
        out_shape=jax.ShapeDtypeStruct((N, D), jnp.float32),
    )(x, W0)

    rows3 = _pad_edges(edge_index[0], jnp.int32)
    cols3 = _pad_edges(edge_index[1], jnp.int32)
    vals3 = _pad_edges(edge_values, jnp.float32)
    zeros = jnp.zeros((N, D), jnp.float32)
    parts = _sc_scatter(pre, rows3, cols3, vals3, zeros)

    return pl.pallas_call(
        _fin_body,
        grid=(N // _MM_ROWS,),
        in_specs=[
            pl.BlockSpec((NC, _MM_ROWS, D), lambda i: (0, i, 0)),
            pl.BlockSpec((1, D), lambda i: (0, 0)),
        ],
        out_specs=pl.BlockSpec((_MM_ROWS, D), lambda i: (i, 0)),
        out_shape=jax.ShapeDtypeStruct((N, D), jnp.float32),
    )(parts, b.reshape(1, D))


# speedup vs baseline: 7.0359x; 1.4789x over previous
import functools

import jax
import jax.numpy as jnp
from jax import lax
from jax.experimental import pallas as pl
from jax.experimental.pallas import tpu as pltpu
from jax.experimental.pallas import tpu_sc as plsc

N = 10000
E = 320000
D = 128

NC = 2
NS = 16
NW = NC * NS
EPW = E // NW
CHUNK = 64
NCHUNK = -(-EPW // CHUNK)
EPW_PAD = NCHUNK * CHUNK
ROWS_PT = N // NS


def _sc_body(pre_hbm, rows_hbm, cols_hbm, vals_hbm, zeros_hbm, out_hbm,
             accum, rows_e, cols_e, vals_e, gb0, gb1,
             cidx0, cidx1, ridx0, vbuf0, gsem0, gsem1):
    c = lax.axis_index("c")
    s = lax.axis_index("s")
    w = c * NS + s

    r0 = pl.multiple_of(s * ROWS_PT, 8)
    pltpu.sync_copy(zeros_hbm.at[pl.ds(r0, ROWS_PT), :],
                    accum.at[pl.ds(r0, ROWS_PT), :])

    pltpu.sync_copy(rows_hbm.at[w], rows_e)
    pltpu.sync_copy(cols_hbm.at[w], cols_e)
    pltpu.sync_copy(vals_hbm.at[w], vals_e)
    plsc.subcore_barrier()

    def copy_chunk(src2d, j, dst):
        for q in range(CHUNK // 16):
            sl = pl.ds(q * 16, 16)
            dst[sl] = src2d[j, sl]

    def scale(buf, vbuf):
        def edge_body(e, carry):
            ev = plsc.load_gather(vbuf, [jnp.full((16,), e, jnp.int32)])
            for q in range(D // 16):
                sl = pl.ds(q * 16, 16)
                buf[e, sl] = buf[e, sl] * ev
            return carry
        lax.fori_loop(0, CHUNK, edge_body, 0, unroll=4)

    def gather_start(j, cidx, buf, sem):
        copy_chunk(cols_e, j, cidx)
        pltpu.async_copy(pre_hbm.at[cidx], buf, sem)

    def finish_chunk(j, cidx, buf, sem):
        pltpu.make_async_copy(pre_hbm.at[cidx], buf, sem).wait()
        copy_chunk(rows_e, j, ridx0)
        copy_chunk(vals_e, j, vbuf0)
        scale(buf, vbuf0)
        pltpu.sync_copy(buf, accum.at[ridx0], add=True)

    gather_start(0, cidx0, gb0, gsem0)

    @pl.loop(0, NCHUNK, step=2)
    def pipeline(j):
        @pl.when(j + 1 < NCHUNK)
        def _():
            gather_start(j + 1, cidx1, gb1, gsem1)

        finish_chunk(j, cidx0, gb0, gsem0)

        @pl.when(j + 1 < NCHUNK)
        def _():
            @pl.when(j + 2 < NCHUNK)
            def _():
                gather_start(j + 2, cidx0, gb0, gsem0)

            finish_chunk(j + 1, cidx1, gb1, gsem1)

    plsc.subcore_barrier()
    pltpu.sync_copy(accum.at[pl.ds(r0, ROWS_PT), :],
                    out_hbm.at[c, pl.ds(r0, ROWS_PT), :])


_sc_scatter = functools.partial(
    pl.kernel,
    out_type=jax.ShapeDtypeStruct((NC, N, D), jnp.float32),
    mesh=plsc.VectorSubcoreMesh(core_axis_name="c", subcore_axis_name="s"),
    scratch_types=[
        pltpu.VMEM_SHARED((N, D), jnp.float32),
        pltpu.VMEM((NCHUNK, CHUNK), jnp.int32),
        pltpu.VMEM((NCHUNK, CHUNK), jnp.int32),
        pltpu.VMEM((NCHUNK, CHUNK), jnp.float32),
        pltpu.VMEM((CHUNK, D), jnp.float32),
        pltpu.VMEM((CHUNK, D), jnp.float32),
        pltpu.VMEM((CHUNK,), jnp.int32),
        pltpu.VMEM((CHUNK,), jnp.int32),
        pltpu.VMEM((CHUNK,), jnp.int32),
        pltpu.VMEM((CHUNK,), jnp.float32),
        pltpu.SemaphoreType.DMA,
        pltpu.SemaphoreType.DMA,
    ],
    compiler_params=pltpu.CompilerParams(use_tc_tiling_on_sc=False,
                                         needs_layout_passes=False),
)(_sc_body)


def _mm_body(x_ref, w_ref, o_ref):
    o_ref[...] = jnp.dot(x_ref[...], w_ref[...],
                         preferred_element_type=jnp.float32)


def _fin_body(p_ref, b_ref, o_ref):
    o_ref[...] = jnp.maximum(p_ref[0] + p_ref[1] + b_ref[...], 0.0)


_MM_ROWS = 1000


def _pad_edges(a, dtype):
    a = a.reshape(NW, EPW)
    pad = jnp.zeros((NW, EPW_PAD - EPW), dtype)
    return jnp.concatenate([a, pad], axis=1).reshape(NW, NCHUNK, CHUNK)


def kernel(x, edge_index, edge_values, W0, b):
    pre = pl.pallas_call(
        _mm_body,
        grid=(N // _MM_ROWS,),
        in_specs=[
            pl.BlockSpec((_MM_ROWS, D), lambda i: (i, 0)),
            pl.BlockSpec((D, D), lambda i: (0, 0)),
        ],
        out_specs=pl.BlockSpec((_MM_ROWS, D), lambda i: (i, 0)),
        out_shape=jax.ShapeDtypeStruct((N, D), jnp.float32),
    )(x, W0)

    rows3 = _pad_edges(edge_index[0], jnp.int32)
    cols3 = _pad_edges(edge_index[1], jnp.int32)
    vals3 = _pad_edges(edge_values, jnp.float32)
    zeros = jnp.zeros((N, D), jnp.float32)
    parts = _sc_scatter(pre, rows3, cols3, vals3, zeros)

    return pl.pallas_call(
        _fin_body,
        grid=(N // _MM_ROWS,),
        in_specs=[
            pl.BlockSpec((NC, _MM_ROWS, D), lambda i: (0, i, 0)),
            pl.BlockSpec((1, D), lambda i: (0, 0)),
        ],
        out_specs=pl.BlockSpec((_MM_ROWS, D), lambda i: (i, 0)),
        out_shape=jax.ShapeDtypeStruct((N, D), jnp.float32),
    )(parts, b.reshape(1, D))


# speedup vs baseline: 9.1285x; 1.2974x over previous
import functools

import jax
import jax.numpy as jnp
from jax import lax
from jax.experimental import pallas as pl
from jax.experimental.pallas import tpu as pltpu
from jax.experimental.pallas import tpu_sc as plsc

N = 10000
E = 320000
D = 128

NC = 2
NS = 16
NW = NC * NS
EPW = E // NW
CHUNK = 80
NCHUNK = -(-EPW // CHUNK)
EPW_PAD = NCHUNK * CHUNK
ROWS_PT = N // NS


def _sc_body(pre_hbm, rows_hbm, cols_hbm, vals_hbm, zeros_hbm, out_hbm,
             accum, rows_e, cols_e, vals_e, gb0, gb1,
             cidx0, cidx1, ridx0, vbuf0, gsem0, gsem1):
    c = lax.axis_index("c")
    s = lax.axis_index("s")
    w = c * NS + s

    r0 = pl.multiple_of(s * ROWS_PT, 8)
    pltpu.sync_copy(zeros_hbm.at[pl.ds(r0, ROWS_PT), :],
                    accum.at[pl.ds(r0, ROWS_PT), :])

    pltpu.sync_copy(rows_hbm.at[w], rows_e)
    pltpu.sync_copy(cols_hbm.at[w], cols_e)
    pltpu.sync_copy(vals_hbm.at[w], vals_e)
    plsc.subcore_barrier()

    def copy_chunk(src2d, j, dst):
        for q in range(CHUNK // 16):
            sl = pl.ds(q * 16, 16)
            dst[sl] = src2d[j, sl]

    def scale(buf, vbuf):
        def edge_body(e, carry):
            ev = plsc.load_gather(vbuf, [jnp.full((16,), e, jnp.int32)])
            for q in range(D // 16):
                sl = pl.ds(q * 16, 16)
                buf[e, sl] = buf[e, sl] * ev
            return carry
        lax.fori_loop(0, CHUNK, edge_body, 0, unroll=4)

    def gather_start(j, cidx, buf, sem):
        copy_chunk(cols_e, j, cidx)
        pltpu.async_copy(pre_hbm.at[cidx], buf, sem)

    def finish_chunk(j, cidx, buf, sem):
        pltpu.make_async_copy(pre_hbm.at[cidx], buf, sem).wait()
        copy_chunk(rows_e, j, ridx0)
        copy_chunk(vals_e, j, vbuf0)
        scale(buf, vbuf0)
        pltpu.sync_copy(buf, accum.at[ridx0], add=True)

    gather_start(0, cidx0, gb0, gsem0)

    @pl.loop(0, NCHUNK, step=2)
    def pipeline(j):
        @pl.when(j + 1 < NCHUNK)
        def _():
            gather_start(j + 1, cidx1, gb1, gsem1)

        finish_chunk(j, cidx0, gb0, gsem0)

        @pl.when(j + 1 < NCHUNK)
        def _():
            @pl.when(j + 2 < NCHUNK)
            def _():
                gather_start(j + 2, cidx0, gb0, gsem0)

            finish_chunk(j + 1, cidx1, gb1, gsem1)

    plsc.subcore_barrier()
    pltpu.sync_copy(accum.at[pl.ds(r0, ROWS_PT), :],
                    out_hbm.at[c, pl.ds(r0, ROWS_PT), :])


_sc_scatter = functools.partial(
    pl.kernel,
    out_type=jax.ShapeDtypeStruct((NC, N, D), jnp.float32),
    mesh=plsc.VectorSubcoreMesh(core_axis_name="c", subcore_axis_name="s"),
    scratch_types=[
        pltpu.VMEM_SHARED((N, D), jnp.float32),
        pltpu.VMEM((NCHUNK, CHUNK), jnp.int32),
        pltpu.VMEM((NCHUNK, CHUNK), jnp.int32),
        pltpu.VMEM((NCHUNK, CHUNK), jnp.float32),
        pltpu.VMEM((CHUNK, D), jnp.float32),
        pltpu.VMEM((CHUNK, D), jnp.float32),
        pltpu.VMEM((CHUNK,), jnp.int32),
        pltpu.VMEM((CHUNK,), jnp.int32),
        pltpu.VMEM((CHUNK,), jnp.int32),
        pltpu.VMEM((CHUNK,), jnp.float32),
        pltpu.SemaphoreType.DMA,
        pltpu.SemaphoreType.DMA,
    ],
    compiler_params=pltpu.CompilerParams(use_tc_tiling_on_sc=False,
                                         needs_layout_passes=False),
)(_sc_body)


def _mm_body(x_ref, w_ref, o_ref):
    o_ref[...] = jnp.dot(x_ref[...], w_ref[...],
                         preferred_element_type=jnp.float32)


def _fin_body(p_ref, b_ref, o_ref):
    o_ref[...] = jnp.maximum(p_ref[0] + p_ref[1] + b_ref[...], 0.0)


_MM_ROWS = 1000


def _pad_edges(a, dtype):
    a = a.reshape(NW, EPW)
    pad = jnp.zeros((NW, EPW_PAD - EPW), dtype)
    return jnp.concatenate([a, pad], axis=1).reshape(NW, NCHUNK, CHUNK)


def kernel(x, edge_index, edge_values, W0, b):
    pre = pl.pallas_call(
        _mm_body,
        grid=(N // _MM_ROWS,),
        in_specs=[
            pl.BlockSpec((_MM_ROWS, D), lambda i: (i, 0)),
            pl.BlockSpec((D, D), lambda i: (0, 0)),
        ],
        out_specs=pl.BlockSpec((_MM_ROWS, D), lambda i: (i, 0)),
        out_shape=jax.ShapeDtypeStruct((N, D), jnp.float32),
    )(x, W0)

    rows3 = _pad_edges(edge_index[0], jnp.int32)
    cols3 = _pad_edges(edge_index[1], jnp.int32)
    vals3 = _pad_edges(edge_values, jnp.float32)
    zeros = jnp.zeros((N, D), jnp.float32)
    parts = _sc_scatter(pre, rows3, cols3, vals3, zeros)

    return pl.pallas_call(
        _fin_body,
        grid=(N // _MM_ROWS,),
        in_specs=[
            pl.BlockSpec((NC, _MM_ROWS, D), lambda i: (0, i, 0)),
            pl.BlockSpec((1, D), lambda i: (0, 0)),
        ],
        out_specs=pl.BlockSpec((_MM_ROWS, D), lambda i: (i, 0)),
        out_shape=jax.ShapeDtypeStruct((N, D), jnp.float32),
    )(parts, b.reshape(1, D))


# speedup vs baseline: 9.1339x; 1.0006x over previous
import functools

import jax
import jax.numpy as jnp
from jax import lax
from jax.experimental import pallas as pl
from jax.experimental.pallas import tpu as pltpu
from jax.experimental.pallas import tpu_sc as plsc

N = 10000
E = 320000
D = 128

NC = 2
NS = 16
NW = NC * NS
EPW = E // NW
CHUNK = 80
NCHUNK = -(-EPW // CHUNK)
EPW_PAD = NCHUNK * CHUNK
ROWS_PT = N // NS


def _sc_body(pre_hbm, rows_hbm, cols_hbm, vals_hbm, zeros_hbm, out_hbm,
             accum, rows_e, cols_e, vals_e, gb0, gb1,
             cidx0, cidx1, ridx0, vbuf0, gsem0, gsem1):
    c = lax.axis_index("c")
    s = lax.axis_index("s")
    w = c * NS + s

    r0 = pl.multiple_of(s * ROWS_PT, 8)
    pltpu.sync_copy(zeros_hbm.at[pl.ds(r0, ROWS_PT), :],
                    accum.at[pl.ds(r0, ROWS_PT), :])

    pltpu.sync_copy(rows_hbm.at[w], rows_e)
    pltpu.sync_copy(cols_hbm.at[w], cols_e)
    pltpu.sync_copy(vals_hbm.at[w], vals_e)
    plsc.subcore_barrier()

    def copy_chunk(src2d, j, dst):
        for q in range(CHUNK // 16):
            sl = pl.ds(q * 16, 16)
            dst[sl] = src2d[j, sl]

    def scale(buf, vbuf):
        def edge_body(e, carry):
            ev = plsc.load_gather(vbuf, [jnp.full((16,), e, jnp.int32)])
            for q in range(D // 16):
                sl = pl.ds(q * 16, 16)
                buf[e, sl] = buf[e, sl] * ev
            return carry
        lax.fori_loop(0, CHUNK, edge_body, 0, unroll=8)

    def gather_start(j, cidx, buf, sem):
        copy_chunk(cols_e, j, cidx)
        pltpu.async_copy(pre_hbm.at[cidx], buf, sem)

    def finish_chunk(j, cidx, buf, sem):
        pltpu.make_async_copy(pre_hbm.at[cidx], buf, sem).wait()
        copy_chunk(rows_e, j, ridx0)
        copy_chunk(vals_e, j, vbuf0)
        scale(buf, vbuf0)
        pltpu.sync_copy(buf, accum.at[ridx0], add=True)

    gather_start(0, cidx0, gb0, gsem0)

    @pl.loop(0, NCHUNK, step=2)
    def pipeline(j):
        @pl.when(j + 1 < NCHUNK)
        def _():
            gather_start(j + 1, cidx1, gb1, gsem1)

        finish_chunk(j, cidx0, gb0, gsem0)

        @pl.when(j + 1 < NCHUNK)
        def _():
            @pl.when(j + 2 < NCHUNK)
            def _():
                gather_start(j + 2, cidx0, gb0, gsem0)

            finish_chunk(j + 1, cidx1, gb1, gsem1)

    plsc.subcore_barrier()
    pltpu.sync_copy(accum.at[pl.ds(r0, ROWS_PT), :],
                    out_hbm.at[c, pl.ds(r0, ROWS_PT), :])


_sc_scatter = functools.partial(
    pl.kernel,
    out_type=jax.ShapeDtypeStruct((NC, N, D), jnp.float32),
    mesh=plsc.VectorSubcoreMesh(core_axis_name="c", subcore_axis_name="s"),
    scratch_types=[
        pltpu.VMEM_SHARED((N, D), jnp.float32),
        pltpu.VMEM((NCHUNK, CHUNK), jnp.int32),
        pltpu.VMEM((NCHUNK, CHUNK), jnp.int32),
        pltpu.VMEM((NCHUNK, CHUNK), jnp.float32),
        pltpu.VMEM((CHUNK, D), jnp.float32),
        pltpu.VMEM((CHUNK, D), jnp.float32),
        pltpu.VMEM((CHUNK,), jnp.int32),
        pltpu.VMEM((CHUNK,), jnp.int32),
        pltpu.VMEM((CHUNK,), jnp.int32),
        pltpu.VMEM((CHUNK,), jnp.float32),
        pltpu.SemaphoreType.DMA,
        pltpu.SemaphoreType.DMA,
    ],
    compiler_params=pltpu.CompilerParams(use_tc_tiling_on_sc=False,
                                         needs_layout_passes=False),
)(_sc_body)


def _mm_body(x_ref, w_ref, o_ref):
    o_ref[...] = jnp.dot(x_ref[...], w_ref[...],
                         preferred_element_type=jnp.float32)


def _fin_body(p_ref, b_ref, o_ref):
    o_ref[...] = jnp.maximum(p_ref[0] + p_ref[1] + b_ref[...], 0.0)


_MM_ROWS = 1000


def _pad_edges(a, dtype):
    a = a.reshape(NW, EPW)
    pad = jnp.zeros((NW, EPW_PAD - EPW), dtype)
    return jnp.concatenate([a, pad], axis=1).reshape(NW, NCHUNK, CHUNK)


def kernel(x, edge_index, edge_values, W0, b):
    pre = pl.pallas_call(
        _mm_body,
        grid=(N // _MM_ROWS,),
        in_specs=[
            pl.BlockSpec((_MM_ROWS, D), lambda i: (i, 0)),
            pl.BlockSpec((D, D), lambda i: (0, 0)),
        ],
        out_specs=pl.BlockSpec((_MM_ROWS, D), lambda i: (i, 0)),
        out_shape=jax.ShapeDtypeStruct((N, D), jnp.float32),
    )(x, W0)

    rows3 = _pad_edges(edge_index[0], jnp.int32)
    cols3 = _pad_edges(edge_index[1], jnp.int32)
    vals3 = _pad_edges(edge_values, jnp.float32)
    zeros = jnp.zeros((N, D), jnp.float32)
    parts = _sc_scatter(pre, rows3, cols3, vals3, zeros)

    return pl.pallas_call(
        _fin_body,
        grid=(N // _MM_ROWS,),
        in_specs=[
            pl.BlockSpec((NC, _MM_ROWS, D), lambda i: (0, i, 0)),
            pl.BlockSpec((1, D), lambda i: (0, 0)),
        ],
        out_specs=pl.BlockSpec((_MM_ROWS, D), lambda i: (i, 0)),
        out_shape=jax.ShapeDtypeStruct((N, D), jnp.float32),
    )(parts, b.reshape(1, D))
